# baseline (speedup 1.0000x reference)
import jax
import jax.numpy as jnp
from jax.experimental import pallas as pl
from jax.experimental.pallas import tpu as pltpu

B, T, I, H, K = 512, 100, 128, 256, 8


def _sigmoid(x):
    return 0.5 + 0.5 * jnp.tanh(0.5 * x)


def _gru_cell(x_bf, h_prev, wih, whh, brz, bin_, bhn, hd):
    gi = jnp.dot(x_bf, wih, preferred_element_type=jnp.float32)
    gh = jnp.dot(h_prev.astype(jnp.bfloat16), whh,
                 preferred_element_type=jnp.float32)
    rz = _sigmoid(gi[:, : 2 * hd] + gh[:, : 2 * hd] + brz)
    r = rz[:, :hd]
    zg = rz[:, hd:]
    n = jnp.tanh(gi[:, 2 * hd:] + bin_ + r * (gh[:, 2 * hd:] + bhn))
    return (1.0 - zg) * n + zg * h_prev


def _fused_kernel(x_ref, c_ref,
                  wih0, whh0, brz0, bin0, bhn0,
                  wih1, whh1, brz1, bin1, bhn1,
                  wih2, whh2, brz2, bin2, bhn2,
                  wih3, whh3, brz3, bin3, bhn3,
                  xbar_ref, z_ref, q_ref,
                  h1, h2, h3, h4):
    t = pl.program_id(0)

    @pl.when(t == 0)
    def _init():
        h1[...] = jnp.zeros_like(h1)
        h2[...] = jnp.zeros_like(h2)
        h3[...] = jnp.zeros_like(h3)
        h4[...] = jnp.zeros_like(h4)

    HB = B // 2
    w0 = wih0[...]
    w0h = whh0[...]
    w1 = wih1[...]
    w1h = whh1[...]
    w2 = wih2[...]
    w2h = whh2[...]
    w3 = wih3[...]
    w3h = whh3[...]
    nh2s = []
    for s in (slice(0, HB), slice(HB, B)):
        x_t = x_ref[s, :]
        nh1 = _gru_cell(x_t, h1[s, :], w0, w0h, brz0[...], bin0[...], bhn0[...], H)
        h1[s, :] = nh1
        nh2 = _gru_cell(nh1.astype(jnp.bfloat16), h2[s, :], w1, w1h,
                        brz1[...], bin1[...], bhn1[...], H)
        h2[s, :] = nh2
        nh3 = _gru_cell(nh2.astype(jnp.bfloat16), h3[s, :], w2, w2h,
                        brz2[...], bin2[...], bhn2[...], I)
        h3[s, :] = nh3
        nh4 = _gru_cell(nh3.astype(jnp.bfloat16), h4[s, :], w3, w3h,
                        brz3[...], bin3[...], bhn3[...], I)
        h4[s, :] = nh4
        xbar_ref[s, :] = nh4
        nh2s.append(nh2)

    @pl.when(t == T - 1)
    def _final():
        z = jnp.concatenate(nh2s, axis=0)
        z_ref[...] = z
        cols = []
        for k in range(K):
            d = z - c_ref[k, :]
            cols.append(jnp.sum(d * d, axis=1, keepdims=True))
        d2 = jnp.concatenate(cols, axis=1)
        qu = 1.0 / (1.0 + d2)
        q_ref[...] = qu / jnp.sum(qu, axis=1, keepdims=True)


def _prep_layer(Wih, Whh, bih, bhh, hd):
    wih = Wih.T.astype(jnp.bfloat16)
    whh = Whh.T.astype(jnp.bfloat16)
    brz = (bih[: 2 * hd] + bhh[: 2 * hd]).reshape(1, 2 * hd)
    bin_ = bih[2 * hd:].reshape(1, hd)
    bhn = bhh[2 * hd:].reshape(1, hd)
    return wih, whh, brz, bin_, bhn


def kernel(x, enc_Wih0, enc_Whh0, enc_bih0, enc_bhh0, enc_Wih1, enc_Whh1,
           enc_bih1, enc_bhh1, dec_Wih0, dec_Whh0, dec_bih0, dec_bhh0,
           dec_Wih1, dec_Whh1, dec_bih1, dec_bhh1, fc_w, fc_b, cluster):
    del fc_w, fc_b
    l0 = _prep_layer(enc_Wih0, enc_Whh0, enc_bih0, enc_bhh0, H)
    l1 = _prep_layer(enc_Wih1, enc_Whh1, enc_bih1, enc_bhh1, H)
    l2 = _prep_layer(dec_Wih0, dec_Whh0, dec_bih0, dec_bhh0, I)
    l3 = _prep_layer(dec_Wih1, dec_Whh1, dec_bih1, dec_bhh1, I)

    x2 = x.reshape(B, T * I).astype(jnp.bfloat16)

    def whole(shape):
        return pl.BlockSpec(shape, lambda t: (0, 0))

    in_specs = [
        pl.BlockSpec((B, I), lambda t: (0, t)),
        whole((K, H)),
    ]
    for (wih, whh, brz, bin_, bhn) in (l0, l1, l2, l3):
        in_specs += [whole(wih.shape), whole(whh.shape), whole(brz.shape),
                     whole(bin_.shape), whole(bhn.shape)]

    out_specs = [
        pl.BlockSpec((B, I), lambda t: (0, t)),
        whole((B, H)),
        whole((B, K)),
    ]
    out_shape = [
        jax.ShapeDtypeStruct((B, T * I), jnp.float32),
        jax.ShapeDtypeStruct((B, H), jnp.float32),
        jax.ShapeDtypeStruct((B, K), jnp.float32),
    ]

    xbar, z, q = pl.pallas_call(
        _fused_kernel,
        grid=(T,),
        in_specs=in_specs,
        out_specs=out_specs,
        out_shape=out_shape,
        scratch_shapes=[
            pltpu.VMEM((B, H), jnp.float32),
            pltpu.VMEM((B, H), jnp.float32),
            pltpu.VMEM((B, I), jnp.float32),
            pltpu.VMEM((B, I), jnp.float32),
        ],
        compiler_params=pltpu.CompilerParams(
            dimension_semantics=("arbitrary",),
        ),
    )(x2, cluster, *l0, *l1, *l2, *l3)

    return (z, xbar.reshape(B, T, I), q)


# speedup vs baseline: 1.1645x; 1.1645x over previous
import jax
import jax.numpy as jnp
from jax.experimental import pallas as pl
from jax.experimental.pallas import tpu as pltpu

B, T, I, H, K = 512, 100, 128, 256, 8


def _sigmoid(x):
    return 0.5 + 0.5 * jnp.tanh(0.5 * x)


def _gru_cell(x_bf, h_prev, wih, whh, brz, bin_, bhn, hd):
    gi = jnp.dot(x_bf, wih, preferred_element_type=jnp.float32)
    gh = jnp.dot(h_prev.astype(jnp.bfloat16), whh,
                 preferred_element_type=jnp.float32)
    rz = _sigmoid(gi[:, : 2 * hd] + gh[:, : 2 * hd] + brz)
    r = rz[:, :hd]
    zg = rz[:, hd:]
    n = jnp.tanh(gi[:, 2 * hd:] + bin_ + r * (gh[:, 2 * hd:] + bhn))
    return (1.0 - zg) * n + zg * h_prev


def _fused_kernel(x_ref, c_ref,
                  wih0, whh0, brz0, bin0, bhn0,
                  wih1, whh1, brz1, bin1, bhn1,
                  wih2, whh2, brz2, bin2, bhn2,
                  wih3, whh3, brz3, bin3, bhn3,
                  xbar_ref, z_ref, q_ref,
                  h1, h2, h3, h4):
    t = pl.program_id(0)

    @pl.when(t == 0)
    def _init():
        h1[...] = jnp.zeros_like(h1)
        h2[...] = jnp.zeros_like(h2)
        h3[...] = jnp.zeros_like(h3)
        h4[...] = jnp.zeros_like(h4)

    x_t = x_ref[...]
    nh1 = _gru_cell(x_t, h1[...], wih0[...], whh0[...], brz0[...], bin0[...],
                    bhn0[...], H)
    h1[...] = nh1
    nh2 = _gru_cell(nh1.astype(jnp.bfloat16), h2[...], wih1[...], whh1[...],
                    brz1[...], bin1[...], bhn1[...], H)
    h2[...] = nh2
    nh3 = _gru_cell(nh2.astype(jnp.bfloat16), h3[...], wih2[...], whh2[...],
                    brz2[...], bin2[...], bhn2[...], I)
    h3[...] = nh3
    nh4 = _gru_cell(nh3.astype(jnp.bfloat16), h4[...], wih3[...], whh3[...],
                    brz3[...], bin3[...], bhn3[...], I)
    h4[...] = nh4
    xbar_ref[...] = nh4

    @pl.when(t == T - 1)
    def _final():
        z = nh2
        z_ref[...] = z
        cols = []
        for k in range(K):
            d = z - c_ref[k, :]
            cols.append(jnp.sum(d * d, axis=1, keepdims=True))
        d2 = jnp.concatenate(cols, axis=1)
        qu = 1.0 / (1.0 + d2)
        q_ref[...] = qu / jnp.sum(qu, axis=1, keepdims=True)


def _prep_layer(Wih, Whh, bih, bhh, hd):
    wih = Wih.T.astype(jnp.bfloat16)
    whh = Whh.T.astype(jnp.bfloat16)
    brz = (bih[: 2 * hd] + bhh[: 2 * hd]).reshape(1, 2 * hd)
    bin_ = bih[2 * hd:].reshape(1, hd)
    bhn = bhh[2 * hd:].reshape(1, hd)
    return wih, whh, brz, bin_, bhn


def kernel(x, enc_Wih0, enc_Whh0, enc_bih0, enc_bhh0, enc_Wih1, enc_Whh1,
           enc_bih1, enc_bhh1, dec_Wih0, dec_Whh0, dec_bih0, dec_bhh0,
           dec_Wih1, dec_Whh1, dec_bih1, dec_bhh1, fc_w, fc_b, cluster):
    del fc_w, fc_b
    l0 = _prep_layer(enc_Wih0, enc_Whh0, enc_bih0, enc_bhh0, H)
    l1 = _prep_layer(enc_Wih1, enc_Whh1, enc_bih1, enc_bhh1, H)
    l2 = _prep_layer(dec_Wih0, dec_Whh0, dec_bih0, dec_bhh0, I)
    l3 = _prep_layer(dec_Wih1, dec_Whh1, dec_bih1, dec_bhh1, I)

    x2 = x.reshape(B, T * I).astype(jnp.bfloat16)

    def whole(shape):
        return pl.BlockSpec(shape, lambda t: (0, 0))

    in_specs = [
        pl.BlockSpec((B, I), lambda t: (0, t)),
        whole((K, H)),
    ]
    for (wih, whh, brz, bin_, bhn) in (l0, l1, l2, l3):
        in_specs += [whole(wih.shape), whole(whh.shape), whole(brz.shape),
                     whole(bin_.shape), whole(bhn.shape)]

    out_specs = [
        pl.BlockSpec((B, I), lambda t: (0, t)),
        whole((B, H)),
        whole((B, K)),
    ]
    out_shape = [
        jax.ShapeDtypeStruct((B, T * I), jnp.float32),
        jax.ShapeDtypeStruct((B, H), jnp.float32),
        jax.ShapeDtypeStruct((B, K), jnp.float32),
    ]

    xbar, z, q = pl.pallas_call(
        _fused_kernel,
        grid=(T,),
        in_specs=in_specs,
        out_specs=out_specs,
        out_shape=out_shape,
        scratch_shapes=[
            pltpu.VMEM((B, H), jnp.float32),
            pltpu.VMEM((B, H), jnp.float32),
            pltpu.VMEM((B, I), jnp.float32),
            pltpu.VMEM((B, I), jnp.float32),
        ],
        compiler_params=pltpu.CompilerParams(
            dimension_semantics=("arbitrary",),
        ),
    )(x2, cluster, *l0, *l1, *l2, *l3)

    return (z, xbar.reshape(B, T, I), q)


# speedup vs baseline: 1.1940x; 1.0253x over previous
import jax
import jax.numpy as jnp
from jax.experimental import pallas as pl
from jax.experimental.pallas import tpu as pltpu

B, T, I, H, K = 512, 100, 128, 256, 8


def _sigmoid(x):
    return 0.5 + 0.5 * jnp.tanh(0.5 * x)


def _gru_cell(x_bf, h_prev, wih, whh, brz, bin_, bhn, hd):
    gi = jnp.dot(x_bf, wih, preferred_element_type=jnp.float32)
    gh = jnp.dot(h_prev.astype(jnp.bfloat16), whh,
                 preferred_element_type=jnp.float32)
    rz = _sigmoid(gi[:, : 2 * hd] + gh[:, : 2 * hd] + brz)
    r = rz[:, :hd]
    zg = rz[:, hd:]
    n = jnp.tanh(gi[:, 2 * hd:] + bin_ + r * (gh[:, 2 * hd:] + bhn))
    return (1.0 - zg) * n + zg * h_prev


def _fused_kernel(x_ref, c_ref,
                  wih0, whh0, brz0, bin0, bhn0,
                  wih1, whh1, brz1, bin1, bhn1,
                  wih2, whh2, brz2, bin2, bhn2,
                  wih3, whh3, brz3, bin3, bhn3,
                  xbar_ref, z_ref, q_ref,
                  h1, h2, h3, h4, s1, s2, s3):
    t = pl.program_id(0)

    @pl.when(t == 0)
    def _init():
        h1[...] = jnp.zeros_like(h1)
        h2[...] = jnp.zeros_like(h2)
        h3[...] = jnp.zeros_like(h3)
        h4[...] = jnp.zeros_like(h4)
        s1[...] = jnp.zeros_like(s1)
        s2[...] = jnp.zeros_like(s2)
        s3[...] = jnp.zeros_like(s3)

    nh1 = _gru_cell(x_ref[...], h1[...], wih0[...], whh0[...],
                    brz0[...], bin0[...], bhn0[...], H)
    nh2 = _gru_cell(s1[...], h2[...], wih1[...], whh1[...],
                    brz1[...], bin1[...], bhn1[...], H)
    nh3 = _gru_cell(s2[...], h3[...], wih2[...], whh2[...],
                    brz2[...], bin2[...], bhn2[...], I)
    nh4 = _gru_cell(s3[...], h4[...], wih3[...], whh3[...],
                    brz3[...], bin3[...], bhn3[...], I)

    h1[...] = nh1
    s1[...] = nh1.astype(jnp.bfloat16)

    @pl.when(t >= 1)
    def _w2():
        h2[...] = nh2
        s2[...] = nh2.astype(jnp.bfloat16)

    @pl.when(t >= 2)
    def _w3():
        h3[...] = nh3
        s3[...] = nh3.astype(jnp.bfloat16)

    @pl.when(t >= 3)
    def _w4():
        h4[...] = nh4
        xbar_ref[...] = nh4

    @pl.when(t == T)
    def _final():
        z = nh2
        z_ref[...] = z
        cols = []
        for k in range(K):
            d = z - c_ref[k, :]
            cols.append(jnp.sum(d * d, axis=1, keepdims=True))
        d2 = jnp.concatenate(cols, axis=1)
        qu = 1.0 / (1.0 + d2)
        q_ref[...] = qu / jnp.sum(qu, axis=1, keepdims=True)


def _prep_layer(Wih, Whh, bih, bhh, hd):
    wih = Wih.T.astype(jnp.bfloat16)
    whh = Whh.T.astype(jnp.bfloat16)
    brz = (bih[: 2 * hd] + bhh[: 2 * hd]).reshape(1, 2 * hd)
    bin_ = bih[2 * hd:].reshape(1, hd)
    bhn = bhh[2 * hd:].reshape(1, hd)
    return wih, whh, brz, bin_, bhn


def kernel(x, enc_Wih0, enc_Whh0, enc_bih0, enc_bhh0, enc_Wih1, enc_Whh1,
           enc_bih1, enc_bhh1, dec_Wih0, dec_Whh0, dec_bih0, dec_bhh0,
           dec_Wih1, dec_Whh1, dec_bih1, dec_bhh1, fc_w, fc_b, cluster):
    del fc_w, fc_b
    l0 = _prep_layer(enc_Wih0, enc_Whh0, enc_bih0, enc_bhh0, H)
    l1 = _prep_layer(enc_Wih1, enc_Whh1, enc_bih1, enc_bhh1, H)
    l2 = _prep_layer(dec_Wih0, dec_Whh0, dec_bih0, dec_bhh0, I)
    l3 = _prep_layer(dec_Wih1, dec_Whh1, dec_bih1, dec_bhh1, I)

    x2 = x.reshape(B, T * I).astype(jnp.bfloat16)

    def whole(shape):
        return pl.BlockSpec(shape, lambda t: (0, 0))

    in_specs = [
        pl.BlockSpec((B, I), lambda t: (0, jnp.minimum(t, T - 1))),
        whole((K, H)),
    ]
    for (wih, whh, brz, bin_, bhn) in (l0, l1, l2, l3):
        in_specs += [whole(wih.shape), whole(whh.shape), whole(brz.shape),
                     whole(bin_.shape), whole(bhn.shape)]

    out_specs = [
        pl.BlockSpec((B, I), lambda t: (0, jnp.maximum(t - 3, 0))),
        whole((B, H)),
        whole((B, K)),
    ]
    out_shape = [
        jax.ShapeDtypeStruct((B, T * I), jnp.float32),
        jax.ShapeDtypeStruct((B, H), jnp.float32),
        jax.ShapeDtypeStruct((B, K), jnp.float32),
    ]

    xbar, z, q = pl.pallas_call(
        _fused_kernel,
        grid=(T + 3,),
        in_specs=in_specs,
        out_specs=out_specs,
        out_shape=out_shape,
        scratch_shapes=[
            pltpu.VMEM((B, H), jnp.float32),
            pltpu.VMEM((B, H), jnp.float32),
            pltpu.VMEM((B, I), jnp.float32),
            pltpu.VMEM((B, I), jnp.float32),
            pltpu.VMEM((B, H), jnp.bfloat16),
            pltpu.VMEM((B, H), jnp.bfloat16),
            pltpu.VMEM((B, I), jnp.bfloat16),
        ],
        compiler_params=pltpu.CompilerParams(
            dimension_semantics=("arbitrary",),
        ),
    )(x2, cluster, *l0, *l1, *l2, *l3)

    return (z, xbar.reshape(B, T, I), q)


# speedup vs baseline: 1.2555x; 1.0516x over previous
import jax
import jax.numpy as jnp
from jax.experimental import pallas as pl
from jax.experimental.pallas import tpu as pltpu

B, T, I, H, K = 512, 100, 128, 256, 8


def _sigmoid(x):
    return 0.5 + 0.5 * jnp.tanh(0.5 * x)


def _gru_cell(x_bf, h_prev, wih, whh, brz, bin_, bhn, hd):
    gi = jnp.dot(x_bf, wih, preferred_element_type=jnp.float32)
    gh = jnp.dot(h_prev.astype(jnp.bfloat16), whh,
                 preferred_element_type=jnp.float32)
    rz = _sigmoid(gi[:, : 2 * hd] + gh[:, : 2 * hd] + brz)
    r = rz[:, :hd]
    zg = rz[:, hd:]
    n = jnp.tanh(gi[:, 2 * hd:] + bin_ + r * (gh[:, 2 * hd:] + bhn))
    return (1.0 - zg) * n + zg * h_prev


def _fused_kernel(x_ref, c_ref,
                  wih0, whh0, brz0, bin0, bhn0,
                  wih1, whh1, brz1, bin1, bhn1,
                  wih2, whh2, brz2, bin2, bhn2,
                  wih3, whh3, brz3, bin3, bhn3,
                  xbar_ref, z_ref, q_ref,
                  h1, h2, h3, h4, s1, s2, s3):
    t = pl.program_id(0)

    @pl.when(t == 0)
    def _init():
        h1[...] = jnp.zeros_like(h1)
        h2[...] = jnp.zeros_like(h2)
        h3[...] = jnp.zeros_like(h3)
        h4[...] = jnp.zeros_like(h4)
        s1[...] = jnp.zeros_like(s1)
        s2[...] = jnp.zeros_like(s2)
        s3[...] = jnp.zeros_like(s3)

    nh1 = _gru_cell(x_ref[...].astype(jnp.bfloat16), h1[...], wih0[...], whh0[...],
                    brz0[...], bin0[...], bhn0[...], H)
    nh2 = _gru_cell(s1[...], h2[...], wih1[...], whh1[...],
                    brz1[...], bin1[...], bhn1[...], H)
    nh3 = _gru_cell(s2[...], h3[...], wih2[...], whh2[...],
                    brz2[...], bin2[...], bhn2[...], I)
    nh4 = _gru_cell(s3[...], h4[...], wih3[...], whh3[...],
                    brz3[...], bin3[...], bhn3[...], I)

    h1[...] = nh1
    s1[...] = nh1.astype(jnp.bfloat16)

    @pl.when(t >= 1)
    def _w2():
        h2[...] = nh2
        s2[...] = nh2.astype(jnp.bfloat16)

    @pl.when(t >= 2)
    def _w3():
        h3[...] = nh3
        s3[...] = nh3.astype(jnp.bfloat16)

    @pl.when(t >= 3)
    def _w4():
        h4[...] = nh4
        xbar_ref[...] = nh4

    @pl.when(t == T)
    def _final():
        z = nh2
        z_ref[...] = z
        cols = []
        for k in range(K):
            d = z - c_ref[k, :]
            cols.append(jnp.sum(d * d, axis=1, keepdims=True))
        d2 = jnp.concatenate(cols, axis=1)
        qu = 1.0 / (1.0 + d2)
        q_ref[...] = qu / jnp.sum(qu, axis=1, keepdims=True)


def _prep_layer(Wih, Whh, bih, bhh, hd):
    wih = Wih.T.astype(jnp.bfloat16)
    whh = Whh.T.astype(jnp.bfloat16)
    brz = (bih[: 2 * hd] + bhh[: 2 * hd]).reshape(1, 2 * hd)
    bin_ = bih[2 * hd:].reshape(1, hd)
    bhn = bhh[2 * hd:].reshape(1, hd)
    return wih, whh, brz, bin_, bhn


def kernel(x, enc_Wih0, enc_Whh0, enc_bih0, enc_bhh0, enc_Wih1, enc_Whh1,
           enc_bih1, enc_bhh1, dec_Wih0, dec_Whh0, dec_bih0, dec_bhh0,
           dec_Wih1, dec_Whh1, dec_bih1, dec_bhh1, fc_w, fc_b, cluster):
    del fc_w, fc_b
    l0 = _prep_layer(enc_Wih0, enc_Whh0, enc_bih0, enc_bhh0, H)
    l1 = _prep_layer(enc_Wih1, enc_Whh1, enc_bih1, enc_bhh1, H)
    l2 = _prep_layer(dec_Wih0, dec_Whh0, dec_bih0, dec_bhh0, I)
    l3 = _prep_layer(dec_Wih1, dec_Whh1, dec_bih1, dec_bhh1, I)

    x2 = x.reshape(B, T * I)

    def whole(shape):
        return pl.BlockSpec(shape, lambda t: (0, 0))

    in_specs = [
        pl.BlockSpec((B, I), lambda t: (0, jnp.minimum(t, T - 1))),
        whole((K, H)),
    ]
    for (wih, whh, brz, bin_, bhn) in (l0, l1, l2, l3):
        in_specs += [whole(wih.shape), whole(whh.shape), whole(brz.shape),
                     whole(bin_.shape), whole(bhn.shape)]

    out_specs = [
        pl.BlockSpec((B, I), lambda t: (0, jnp.maximum(t - 3, 0))),
        whole((B, H)),
        whole((B, K)),
    ]
    out_shape = [
        jax.ShapeDtypeStruct((B, T * I), jnp.float32),
        jax.ShapeDtypeStruct((B, H), jnp.float32),
        jax.ShapeDtypeStruct((B, K), jnp.float32),
    ]

    xbar, z, q = pl.pallas_call(
        _fused_kernel,
        grid=(T + 3,),
        in_specs=in_specs,
        out_specs=out_specs,
        out_shape=out_shape,
        scratch_shapes=[
            pltpu.VMEM((B, H), jnp.float32),
            pltpu.VMEM((B, H), jnp.float32),
            pltpu.VMEM((B, I), jnp.float32),
            pltpu.VMEM((B, I), jnp.float32),
            pltpu.VMEM((B, H), jnp.bfloat16),
            pltpu.VMEM((B, H), jnp.bfloat16),
            pltpu.VMEM((B, I), jnp.bfloat16),
        ],
        compiler_params=pltpu.CompilerParams(
            dimension_semantics=("arbitrary",),
        ),
    )(x2, cluster, *l0, *l1, *l2, *l3)

    return (z, xbar.reshape(B, T, I), q)
